# baseline (speedup 1.0000x reference)
import jax
import jax.numpy as jnp
from jax import lax
from jax.experimental import pallas as pl
from jax.experimental.pallas import tpu as pltpu
from jax.experimental.pallas import tpu_sc as plsc

_BATCH = 4096
_VOCAB = 100000
_N_NOISE = 27
_LAMBDA = 0.1

_NC = 2
_NS = 16
_NW = _NC * _NS
_ROWS_PER_W = _BATCH // _NW
_NPAD = 32


def _body(flat_hbm, noise_hbm, out_hbm, nvec_v, idx_v, data_v, acc_v, sem):
    wid = lax.axis_index("s") * _NC + lax.axis_index("c")
    row_base = wid * _ROWS_PER_W

    pltpu.sync_copy(noise_hbm, nvec_v)
    vn0 = nvec_v[pl.ds(0, 16)]
    vn1 = nvec_v[pl.ds(16, 16)]

    def write_row(r, carry):
        off = (row_base + r) * _VOCAB
        idx_v[pl.ds(r * _NPAD, 16)] = vn0 + off
        idx_v[pl.ds(r * _NPAD + 16, 16)] = vn1 + off
        return carry

    lax.fori_loop(0, _ROWS_PER_W, write_row, 0)

    pltpu.async_copy(flat_hbm.at[idx_v], data_v, sem).wait()

    mask = lax.iota(jnp.int32, 16) < (_N_NOISE - 16)

    def red(r, acc):
        a = data_v[pl.ds(r * _NPAD, 16)]
        b = data_v[pl.ds(r * _NPAD + 16, 16)]
        return acc + a + jnp.where(mask, b, 0.0)

    acc = lax.fori_loop(0, _ROWS_PER_W, red, jnp.zeros((16,), jnp.float32))
    acc_v[...] = acc
    pltpu.sync_copy(acc_v, out_hbm.at[pl.ds(wid * 16, 16)])


def kernel(sparse_repr, noise_indices):
    flat = sparse_repr.reshape(-1)
    noise_padded = jnp.zeros((_NPAD,), jnp.int32).at[:_N_NOISE].set(noise_indices)
    partials = pl.kernel(
        _body,
        out_type=jax.ShapeDtypeStruct((_NW * 16,), jnp.float32),
        mesh=plsc.VectorSubcoreMesh(core_axis_name="c", subcore_axis_name="s"),
        scratch_types=[
            pltpu.VMEM((_NPAD,), jnp.int32),
            pltpu.VMEM((_ROWS_PER_W * _NPAD,), jnp.int32),
            pltpu.VMEM((_ROWS_PER_W * _NPAD,), jnp.float32),
            pltpu.VMEM((16,), jnp.float32),
            pltpu.SemaphoreType.DMA,
        ],
    )(flat, noise_padded)
    return (_LAMBDA / _BATCH) * jnp.sum(partials)


# speedup vs baseline: 141.6379x; 141.6379x over previous
import jax
import jax.numpy as jnp
from jax import lax
from jax.experimental import pallas as pl
from jax.experimental.pallas import tpu as pltpu
from jax.experimental.pallas import tpu_sc as plsc

_BATCH = 4096
_VOCAB = 100000
_N_NOISE = 27
_LAMBDA = 0.1

_NC = 2
_NS = 16
_NW = _NC * _NS
_COLS_PER_W = _BATCH // _NW
_NPAD = 32


def _body(rep_hbm, noise_hbm, out_hbm, nvec_v, data_v, acc_v, sem):
    wid = lax.axis_index("s") * _NC + lax.axis_index("c")
    col_base = wid * _COLS_PER_W

    pltpu.sync_copy(noise_hbm, nvec_v.at[pl.ds(0, _N_NOISE)])
    mask = lax.iota(jnp.int32, 16) < (_N_NOISE - 16)
    nvec_v[pl.ds(16, 16)] = jnp.where(mask, nvec_v[pl.ds(16, 16)], 0)

    pltpu.async_copy(
        rep_hbm.at[nvec_v, pl.ds(col_base, _COLS_PER_W)], data_v, sem
    ).wait()

    acc = jnp.zeros((16,), jnp.float32)
    for r in range(_N_NOISE):
        for c in range(_COLS_PER_W // 16):
            acc = acc + data_v[r, pl.ds(c * 16, 16)]
    acc_v[...] = acc
    pltpu.sync_copy(acc_v, out_hbm.at[pl.ds(wid * 16, 16)])


def kernel(sparse_repr, noise_indices):
    rep_t = sparse_repr.T
    partials = pl.kernel(
        _body,
        out_type=jax.ShapeDtypeStruct((_NW * 16,), jnp.float32),
        mesh=plsc.VectorSubcoreMesh(core_axis_name="c", subcore_axis_name="s"),
        compiler_params=pltpu.CompilerParams(use_tc_tiling_on_sc=True),
        scratch_types=[
            pltpu.VMEM((_NPAD,), jnp.int32),
            pltpu.VMEM((_NPAD, _COLS_PER_W), jnp.float32),
            pltpu.VMEM((16,), jnp.float32),
            pltpu.SemaphoreType.DMA,
        ],
    )(rep_t, noise_indices)
    return (_LAMBDA / _BATCH) * jnp.sum(partials)
